# Initial kernel scaffold; baseline (speedup 1.0000x reference)
import jax
import jax.numpy as jnp
from jax.experimental import pallas as pl


def kernel(hidden, mask, conv_w, conv_b, cifout_w, cifout_b, up_w, up_b, cifout2_w, cifout2_b):
    raise NotImplementedError("write your pallas kernel here")


# speedup vs baseline: 16.8942x; 16.8942x over previous
import jax
import jax.numpy as jnp
from jax.experimental import pallas as pl
from jax.experimental.pallas import tpu as pltpu

B, T, D = 16, 512, 512
NPROJ = 8


def _conv_proj_kernel(hp_ref, aw_ref, cb_ref, wp_ref, bp_ref, s_ref, sums_ref):
    x = hp_ref[...]
    x0 = jax.lax.slice(x, (0, 0), (T, D))
    x1 = jax.lax.slice(x, (1, 0), (T + 1, D))
    x2 = jax.lax.slice(x, (2, 0), (T + 2, D))
    out = (jnp.dot(x0, aw_ref[0], preferred_element_type=jnp.float32)
           + jnp.dot(x1, aw_ref[1], preferred_element_type=jnp.float32)
           + jnp.dot(x2, aw_ref[2], preferred_element_type=jnp.float32)
           + cb_ref[...])
    out = jnp.maximum(out, 0.0)
    p = jnp.dot(out, wp_ref[...], preferred_element_type=jnp.float32) + bp_ref[...]
    s = jax.nn.sigmoid(p)
    s_ref[...] = s
    tn = jnp.sum(jax.lax.slice(s, (0, 0), (T, 1)))
    tn2 = jnp.sum(jax.lax.slice(s, (0, 1), (T, 6)))
    li = jax.lax.broadcasted_iota(jnp.int32, (1, 128), 1)
    sums_ref[...] = jnp.where(li == 0, tn, jnp.where(li == 1, tn2, 0.0))


def _scan_kernel(at_ref, fires_ref, w1_ref, w2_ref, s1_ref, s2_ref):
    def body(t, carry):
        integ, n = carry
        a = at_ref[pl.ds(t, 1), :]
        dist = 1.0 - integ
        integ2 = integ + a
        fires_ref[pl.ds(t, 1), :] = integ2
        fire = integ2 >= 1.0
        cur = jnp.where(fire, dist, a)
        w1_ref[pl.ds(t, 1), :] = cur
        w2_ref[pl.ds(t, 1), :] = jnp.where(fire, a - cur, 0.0)
        s1_ref[pl.ds(t, 1), :] = n
        n2 = n + jnp.where(fire, 1.0, 0.0)
        s2_ref[pl.ds(t, 1), :] = n2
        integ3 = jnp.where(fire, integ2 - 1.0, integ2)
        return (integ3, n2)

    zero = jnp.zeros((1, B), jnp.float32)
    _, kfin = jax.lax.fori_loop(0, T, body, (zero, zero))
    w1_ref[...] = jnp.where(s1_ref[...] < kfin, w1_ref[...], 0.0)
    w2_ref[...] = jnp.where(s2_ref[...] < kfin, w2_ref[...], 0.0)


def _seg_gather_kernel(h_ref, w1_ref, w2_ref, s1_ref, s2_ref, out_ref):
    j = jax.lax.broadcasted_iota(jnp.int32, (T, T), 0).astype(jnp.float32)
    m = (jnp.where(s1_ref[...] == j, w1_ref[...], 0.0)
         + jnp.where(s2_ref[...] == j, w2_ref[...], 0.0))
    out_ref[...] = jnp.dot(m, h_ref[...], preferred_element_type=jnp.float32)


def kernel(hidden, mask, conv_w, conv_b, cifout_w, cifout_b, up_w, up_b,
           cifout2_w, cifout2_b):
    f32 = jnp.float32
    hp = jnp.pad(hidden, ((0, 0), (1, 7), (0, 0)))
    aw = jnp.transpose(conv_w, (2, 1, 0))
    cb = conv_b[None, :]
    v = jnp.einsum('o,oik->ik', cifout2_w[0], up_w)
    wp = jnp.concatenate([cifout_w.T, v, jnp.zeros((D, 2), f32)], axis=1)
    c2 = jnp.dot(up_b, cifout2_w[0]) + cifout2_b[0]
    bp = jnp.concatenate([cifout_b, jnp.full((5,), c2, f32),
                          jnp.zeros((2,), f32)])[None, :]

    s, sums = pl.pallas_call(
        _conv_proj_kernel,
        grid=(B,),
        in_specs=[
            pl.BlockSpec((None, T + 8, D), lambda b: (b, 0, 0)),
            pl.BlockSpec((3, D, D), lambda b: (0, 0, 0)),
            pl.BlockSpec((1, D), lambda b: (0, 0)),
            pl.BlockSpec((D, NPROJ), lambda b: (0, 0)),
            pl.BlockSpec((1, NPROJ), lambda b: (0, 0)),
        ],
        out_specs=[
            pl.BlockSpec((None, T, NPROJ), lambda b: (b, 0, 0)),
            pl.BlockSpec((None, 1, 128), lambda b: (b, 0, 0)),
        ],
        out_shape=[
            jax.ShapeDtypeStruct((B, T, NPROJ), f32),
            jax.ShapeDtypeStruct((B, 1, 128), f32),
        ],
        compiler_params=pltpu.CompilerParams(
            dimension_semantics=("parallel",)),
    )(hp, aw, cb, wp, bp)

    alphas = s[:, :, 0]
    token_num = sums[:, 0, 0]
    token_num2 = sums[:, 0, 1]

    at = alphas.T
    fires_t, w1_t, w2_t, s1_t, s2_t = pl.pallas_call(
        _scan_kernel,
        grid=(1,),
        in_specs=[pl.BlockSpec((T, B), lambda i: (0, 0))],
        out_specs=[pl.BlockSpec((T, B), lambda i: (0, 0))] * 5,
        out_shape=[jax.ShapeDtypeStruct((T, B), f32)] * 5,
        compiler_params=pltpu.CompilerParams(
            dimension_semantics=("arbitrary",)),
    )(at)

    cif_peak = fires_t.T
    w1r = w1_t.T.reshape(B, 1, T)
    w2r = w2_t.T.reshape(B, 1, T)
    s1r = s1_t.T.reshape(B, 1, T)
    s2r = s2_t.T.reshape(B, 1, T)

    acoustic = pl.pallas_call(
        _seg_gather_kernel,
        grid=(B,),
        in_specs=[
            pl.BlockSpec((None, T, D), lambda b: (b, 0, 0)),
            pl.BlockSpec((None, 1, T), lambda b: (b, 0, 0)),
            pl.BlockSpec((None, 1, T), lambda b: (b, 0, 0)),
            pl.BlockSpec((None, 1, T), lambda b: (b, 0, 0)),
            pl.BlockSpec((None, 1, T), lambda b: (b, 0, 0)),
        ],
        out_specs=pl.BlockSpec((None, T, D), lambda b: (b, 0, 0)),
        out_shape=jax.ShapeDtypeStruct((B, T, D), f32),
        compiler_params=pltpu.CompilerParams(
            dimension_semantics=("parallel",)),
    )(hidden, w1r, w2r, s1r, s2r)

    return (acoustic, token_num, alphas, cif_peak, token_num2)
